# baseline (speedup 1.0000x reference)
import jax
import jax.numpy as jnp
from jax.experimental import pallas as pl


def _pos_kernel(col_ref, row_ref, out_ref):
    s = pl.program_id(0)
    d, hblk, w = out_ref.shape[1], out_ref.shape[2], out_ref.shape[3]

    @pl.when(s == 0)
    def _col():
        colT = col_ref[...].T
        out_ref[0] = jnp.broadcast_to(colT[:, None, :], (d, hblk, w))

    @pl.when(s == 1)
    def _row():
        rowT = row_ref[...].T
        out_ref[0] = jnp.broadcast_to(rowT[:, :, None], (d, hblk, w))


def kernel(x, row_embed, col_embed):
    B, C, H, W = x.shape
    d = col_embed.shape[1]
    HBLK = 32

    col = col_embed[:W]
    row = row_embed[:H]

    grid = (2, B, H // HBLK)
    out = pl.pallas_call(
        _pos_kernel,
        grid=grid,
        in_specs=[
            pl.BlockSpec((W, d), lambda s, b, h: (0, 0)),
            pl.BlockSpec((HBLK, d), lambda s, b, h: (h, 0)),
        ],
        out_specs=pl.BlockSpec((1, d, HBLK, W), lambda s, b, h: (b, s, h, 0)),
        out_shape=jax.ShapeDtypeStruct((B, C, H, W), x.dtype),
    )(col, row)
    return out
